# Initial kernel scaffold; baseline (speedup 1.0000x reference)
import jax
import jax.numpy as jnp
from jax.experimental import pallas as pl


def kernel(x, adj, W1, b1, g1, beta1, W2, b2, g2, beta2, Wc, bc):
    raise NotImplementedError("write your pallas kernel here")


# speedup vs baseline: 1.0936x; 1.0936x over previous
import functools

import jax
import jax.numpy as jnp
from jax.experimental import pallas as pl
from jax.experimental.pallas import tpu as pltpu


def _layer1_kernel(adj_ref, x_ref, w1_ref, b1_ref, g1_ref, beta1_ref, out_ref):
    s = jnp.dot(adj_ref[...], x_ref[...], preferred_element_type=jnp.float32)
    t = jnp.dot(s, w1_ref[...].T, preferred_element_type=jnp.float32) + b1_ref[...]
    m = jnp.mean(t, axis=-1, keepdims=True)
    v = jnp.mean((t - m) ** 2, axis=-1, keepdims=True)
    h = (t - m) * jax.lax.rsqrt(v + 1e-5) * g1_ref[...] + beta1_ref[...]
    out_ref[...] = jnp.maximum(h, 0.0)


def _layer2_kernel(adj_ref, h1_ref, w2_ref, b2_ref, g2_ref, beta2_ref,
                   wc_ref, bc_ref, out_ref):
    s = jnp.dot(adj_ref[...], h1_ref[...], preferred_element_type=jnp.float32)
    t = jnp.dot(s, w2_ref[...].T, preferred_element_type=jnp.float32) + b2_ref[...]
    m = jnp.mean(t, axis=-1, keepdims=True)
    v = jnp.mean((t - m) ** 2, axis=-1, keepdims=True)
    h = (t - m) * jax.lax.rsqrt(v + 1e-5) * g2_ref[...] + beta2_ref[...]
    h = jnp.maximum(h, 0.0)
    out_ref[...] = jnp.sum(h * wc_ref[...], axis=-1, keepdims=True) + bc_ref[0]


def _pick_block(n: int) -> int:
    for bn in (512, 400, 250, 200, 125, 100, 50, 25, 16, 8):
        if n % bn == 0:
            return bn
    return n


@functools.partial(jax.jit, static_argnames=())
def kernel(x, adj, W1, b1, g1, beta1, W2, b2, g2, beta2, Wc, bc):
    n, d = x.shape
    h_dim = W1.shape[0]
    bn = _pick_block(n)
    grid = (n // bn,)
    params = pltpu.CompilerParams(dimension_semantics=("parallel",))

    adj_spec = pl.BlockSpec((bn, n), lambda i: (i, 0))
    full = lambda shape: pl.BlockSpec(shape, lambda i: (0,) * len(shape))

    h1 = pl.pallas_call(
        _layer1_kernel,
        grid=grid,
        in_specs=[
            adj_spec,
            full((n, d)),
            full(W1.shape),
            full(b1.shape),
            full(g1.shape),
            full(beta1.shape),
        ],
        out_specs=pl.BlockSpec((bn, h_dim), lambda i: (i, 0)),
        out_shape=jax.ShapeDtypeStruct((n, h_dim), jnp.float32),
        compiler_params=params,
    )(adj, x, W1, b1, g1, beta1)

    out = pl.pallas_call(
        _layer2_kernel,
        grid=grid,
        in_specs=[
            adj_spec,
            full((n, h_dim)),
            full(W2.shape),
            full(b2.shape),
            full(g2.shape),
            full(beta2.shape),
            full(Wc.shape),
            full(bc.shape),
        ],
        out_specs=pl.BlockSpec((bn, 1), lambda i: (i, 0)),
        out_shape=jax.ShapeDtypeStruct((n, 1), jnp.float32),
        compiler_params=params,
    )(adj, h1, W2, b2, g2, beta2, Wc, bc)

    return out.reshape(n)


# speedup vs baseline: 1.0944x; 1.0008x over previous
import functools

import jax
import jax.numpy as jnp
from jax.experimental import pallas as pl
from jax.experimental.pallas import tpu as pltpu


def _layer1_kernel(adj_ref, x_ref, w1_ref, b1_ref, g1_ref, beta1_ref, out_ref):
    s = jnp.dot(adj_ref[...], x_ref[...], preferred_element_type=jnp.float32)
    t = jnp.dot(s, w1_ref[...].T, preferred_element_type=jnp.float32) + b1_ref[...]
    m = jnp.mean(t, axis=-1, keepdims=True)
    v = jnp.mean((t - m) ** 2, axis=-1, keepdims=True)
    h = (t - m) * jax.lax.rsqrt(v + 1e-5) * g1_ref[...] + beta1_ref[...]
    out_ref[...] = jnp.maximum(h, 0.0)


def _layer2_kernel(adj_ref, h1_ref, w2_ref, b2_ref, g2_ref, beta2_ref,
                   wc_ref, bc_ref, out_ref):
    s = jnp.dot(adj_ref[...], h1_ref[...], preferred_element_type=jnp.float32)
    t = jnp.dot(s, w2_ref[...].T, preferred_element_type=jnp.float32) + b2_ref[...]
    m = jnp.mean(t, axis=-1, keepdims=True)
    v = jnp.mean((t - m) ** 2, axis=-1, keepdims=True)
    h = (t - m) * jax.lax.rsqrt(v + 1e-5) * g2_ref[...] + beta2_ref[...]
    h = jnp.maximum(h, 0.0)
    out_ref[...] = jnp.sum(h * wc_ref[...], axis=-1, keepdims=True) + bc_ref[0]


def _pick_block(n: int) -> int:
    for bn in (400, 250, 200, 125, 100, 50, 25, 16, 8):
        if n % bn == 0:
            return bn
    return n


@functools.partial(jax.jit, static_argnames=())
def kernel(x, adj, W1, b1, g1, beta1, W2, b2, g2, beta2, Wc, bc):
    n, d = x.shape
    h_dim = W1.shape[0]
    bn = _pick_block(n)
    grid = (n // bn,)
    params = pltpu.CompilerParams(
        dimension_semantics=("parallel",),
        vmem_limit_bytes=64 * 1024 * 1024,
    )

    adj_spec = pl.BlockSpec((bn, n), lambda i: (i, 0))
    full = lambda shape: pl.BlockSpec(shape, lambda i: (0,) * len(shape))

    h1 = pl.pallas_call(
        _layer1_kernel,
        grid=grid,
        in_specs=[
            adj_spec,
            full((n, d)),
            full(W1.shape),
            full(b1.shape),
            full(g1.shape),
            full(beta1.shape),
        ],
        out_specs=pl.BlockSpec((bn, h_dim), lambda i: (i, 0)),
        out_shape=jax.ShapeDtypeStruct((n, h_dim), jnp.float32),
        compiler_params=params,
    )(adj, x, W1, b1, g1, beta1)

    out = pl.pallas_call(
        _layer2_kernel,
        grid=grid,
        in_specs=[
            adj_spec,
            full((n, h_dim)),
            full(W2.shape),
            full(b2.shape),
            full(g2.shape),
            full(beta2.shape),
            full(Wc.shape),
            full(bc.shape),
        ],
        out_specs=pl.BlockSpec((bn, 1), lambda i: (i, 0)),
        out_shape=jax.ShapeDtypeStruct((n, 1), jnp.float32),
        compiler_params=params,
    )(adj, h1, W2, b2, g2, beta2, Wc, bc)

    return out.reshape(n)
